# baseline (speedup 1.0000x reference)
import dataclasses
import functools

import jax
import jax.numpy as jnp
from jax import lax
from jax.experimental import pallas as pl
from jax.experimental.pallas import tpu as pltpu
from jax.experimental.pallas import tpu_sc as plsc

LANES = 16
NUM_SC_CORES = 2
NUM_SUBCORES = 16
CHUNK = 128


def _proj_body(x_ref, wl_ref, bl_ref, p_ref):
    n_rel, d = p_ref.shape[0], p_ref.shape[2]
    x = x_ref[...]
    for k in range(n_rel):
        p_ref[k] = (
            jnp.dot(x, wl_ref[:, k * d:(k + 1) * d],
                    preferred_element_type=jnp.float32)
            + bl_ref[k:k + 1, :]
        )


def _hidden_body(x_ref, ws_ref, bs_ref, h_ref):
    h_ref[...] = (
        jnp.dot(x_ref[...], ws_ref[...], preferred_element_type=jnp.float32)
        + bs_ref[...]
    )


def _combine_body(h_ref, a_ref, o_ref):
    a = a_ref[...]
    o_ref[...] = jnp.maximum(h_ref[...] + a[0] + a[1], 0.0)


def _sc_body(n_nodes, d, n_relations, n_chunks_per_core, p_hbm, ei_hbm,
             typ_hbm, out_hbm, acc, ei_v, typ_v, dst_v, key_v,
             rows_v, zero_v, gsem, isem):
    cid = lax.axis_index("c")
    sid = lax.axis_index("s")

    zrows = zero_v.shape[0]
    n_strips = n_nodes // zrows
    strip_iters = (n_strips + NUM_SUBCORES - 1) // NUM_SUBCORES

    @pl.loop(0, zrows)
    def _(zr):
        @pl.loop(0, d // LANES)
        def _(c):
            zero_v[zr, pl.ds(c * LANES, LANES)] = jnp.zeros((LANES,), jnp.float32)

    @pl.loop(0, strip_iters)
    def _(t):
        strip = sid + t * NUM_SUBCORES

        @pl.when(strip < n_strips)
        def _():
            pltpu.sync_copy(zero_v, acc.at[pl.ds(strip * zrows, zrows)])

    plsc.subcore_barrier()

    max_iters = (n_chunks_per_core + NUM_SUBCORES - 1) // NUM_SUBCORES

    def edge_base(chunk):
        return (cid * n_chunks_per_core + chunk) * CHUNK

    def issue_idx(chunk, h):
        base = edge_base(chunk)
        pltpu.async_copy(ei_hbm.at[pl.ds(2 * base, 2 * CHUNK)],
                         ei_v.at[pl.ds(h * 2 * CHUNK, 2 * CHUNK)], isem)
        pltpu.async_copy(typ_hbm.at[pl.ds(base, CHUNK)], typ_v.at[h], isem)

    def wait_idx_compute_key_issue_gather(chunk, h):
        base = edge_base(chunk)
        pltpu.make_async_copy(ei_hbm.at[pl.ds(2 * base, 2 * CHUNK)],
                              ei_v.at[pl.ds(h * 2 * CHUNK, 2 * CHUNK)], isem).wait()
        pltpu.make_async_copy(typ_hbm.at[pl.ds(base, CHUNK)], typ_v.at[h], isem).wait()

        @pl.loop(0, CHUNK // LANES)
        def _(i):
            sl = pl.ds(i * LANES, LANES)
            evens = (lax.iota(jnp.int32, LANES) + i * LANES) * 2 + h * 2 * CHUNK
            src16 = plsc.load_gather(ei_v, [evens])
            dst_v[h, sl] = plsc.load_gather(ei_v, [evens + 1])
            key_v[h, sl] = typ_v[h, sl] * n_nodes + src16

        pltpu.async_copy(p_hbm.at[key_v.at[h]], rows_v.at[h], gsem)

    def stage(c_cur, h_cur, h_nxt):
        c_nxt = c_cur + NUM_SUBCORES

        @pl.when(c_nxt < n_chunks_per_core)
        def _():
            issue_idx(c_nxt, h_nxt)

        @pl.when(c_cur < n_chunks_per_core)
        def _():
            pltpu.make_async_copy(p_hbm.at[key_v.at[h_cur]],
                                  rows_v.at[h_cur], gsem).wait()

        @pl.when(c_nxt < n_chunks_per_core)
        def _():
            wait_idx_compute_key_issue_gather(c_nxt, h_nxt)

        @pl.when(c_cur < n_chunks_per_core)
        def _():
            pltpu.sync_copy(rows_v.at[h_cur], acc.at[dst_v.at[h_cur]], add=True)

    issue_idx(sid, 0)
    wait_idx_compute_key_issue_gather(sid, 0)

    @pl.loop(0, (max_iters + 1) // 2)
    def _(j):
        stage(sid + (2 * j) * NUM_SUBCORES, 0, 1)
        stage(sid + (2 * j + 1) * NUM_SUBCORES, 1, 0)

    plsc.subcore_barrier()

    @pl.loop(0, strip_iters)
    def _(t):
        strip = sid + t * NUM_SUBCORES

        @pl.when(strip < n_strips)
        def _():
            r0 = strip * zrows
            pltpu.sync_copy(acc.at[pl.ds(r0, zrows)],
                            out_hbm.at[cid, pl.ds(r0, zrows)])


def kernel(input, edge_index, edge_type, self_loop_W, self_loop_b,
           linear_W, linear_b):
    n, d = input.shape
    e = edge_index.shape[0]
    r = linear_W.shape[0] // d

    wl_t = linear_W.T
    ws_t = self_loop_W.T
    bl = linear_b.reshape(r, d)
    bs = self_loop_b.reshape(1, d)
    typ = edge_type.astype(jnp.int32)

    row_blk = 2000
    grid = (n // row_blk,)
    p = pl.pallas_call(
        _proj_body,
        grid=grid,
        in_specs=[
            pl.BlockSpec((row_blk, d), lambda i: (i, 0)),
            pl.BlockSpec((d, r * d), lambda i: (0, 0)),
            pl.BlockSpec((r, d), lambda i: (0, 0)),
        ],
        out_specs=pl.BlockSpec((r, row_blk, d), lambda i: (0, i, 0)),
        out_shape=jax.ShapeDtypeStruct((r, n, d), jnp.float32),
    )(input, wl_t, bl)
    p4 = p.reshape(r * n, d)

    hidden = pl.pallas_call(
        _hidden_body,
        grid=grid,
        in_specs=[
            pl.BlockSpec((row_blk, d), lambda i: (i, 0)),
            pl.BlockSpec((d, d), lambda i: (0, 0)),
            pl.BlockSpec((1, d), lambda i: (0, 0)),
        ],
        out_specs=pl.BlockSpec((row_blk, d), lambda i: (i, 0)),
        out_shape=jax.ShapeDtypeStruct((n, d), jnp.float32),
    )(input, ws_t, bs)

    n_chunks_per_core = e // (NUM_SC_CORES * CHUNK)
    mesh = plsc.VectorSubcoreMesh(core_axis_name="c", subcore_axis_name="s")
    sc_params = pltpu.CompilerParams()
    if "needs_layout_passes" in pltpu.CompilerParams.__dataclass_fields__:
        sc_params = dataclasses.replace(sc_params, needs_layout_passes=False)
    sc_fn = pl.kernel(
        functools.partial(_sc_body, n, d, r, n_chunks_per_core),
        out_type=jax.ShapeDtypeStruct((NUM_SC_CORES, n, d), jnp.float32),
        mesh=mesh,
        scratch_types=[
            pltpu.VMEM_SHARED((n, d), jnp.float32),
            pltpu.VMEM((4 * CHUNK,), jnp.int32),
            pltpu.VMEM((2, CHUNK), jnp.int32),
            pltpu.VMEM((2, CHUNK), jnp.int32),
            pltpu.VMEM((2, CHUNK), jnp.int32),
            pltpu.VMEM((2, CHUNK, d), jnp.float32),
            pltpu.VMEM((80, d), jnp.float32),
            pltpu.SemaphoreType.DMA,
            pltpu.SemaphoreType.DMA,
        ],
        compiler_params=sc_params,
    )
    acc = sc_fn(p4, edge_index.astype(jnp.int32).reshape(-1), typ)

    out = pl.pallas_call(
        _combine_body,
        grid=grid,
        in_specs=[
            pl.BlockSpec((row_blk, d), lambda i: (i, 0)),
            pl.BlockSpec((NUM_SC_CORES, row_blk, d), lambda i: (0, i, 0)),
        ],
        out_specs=pl.BlockSpec((row_blk, d), lambda i: (i, 0)),
        out_shape=jax.ShapeDtypeStruct((n, d), jnp.float32),
    )(hidden, acc)
    return out


# speedup vs baseline: 1.6390x; 1.6390x over previous
import functools

import jax
import jax.numpy as jnp
from jax import lax
from jax.experimental import pallas as pl
from jax.experimental.pallas import tpu as pltpu
from jax.experimental.pallas import tpu_sc as plsc

LANES = 16
NUM_SC_CORES = 2
NUM_SUBCORES = 16
CHUNK = 128


def _proj_body(x_ref, wl_ref, bl_ref, p_ref):
    rr = pl.program_id(1)
    n_rel = bl_ref.shape[0]
    bias = bl_ref[0:1, :]
    for k in range(1, n_rel):
        bias = jnp.where(rr == k, bl_ref[k:k + 1, :], bias)
    p_ref[...] = (
        jnp.dot(x_ref[...], wl_ref[...], preferred_element_type=jnp.float32)
        + bias
    )


def _hidden_body(x_ref, ws_ref, bs_ref, h_ref):
    h_ref[...] = (
        jnp.dot(x_ref[...], ws_ref[...], preferred_element_type=jnp.float32)
        + bs_ref[...]
    )


def _combine_body(h_ref, a_ref, o_ref):
    a = a_ref[...]
    o_ref[...] = jnp.maximum(h_ref[...] + a[0] + a[1], 0.0)


def _sc_body(n_nodes, d, n_relations, n_chunks_per_core, p_hbm, src_hbm,
             typ_hbm, dst_hbm, out_hbm, acc, src_v, typ_v, dst_v, key_v,
             rows_v, zero_v, gsem, isem):
    cid = lax.axis_index("c")
    sid = lax.axis_index("s")

    zrows = zero_v.shape[0]
    n_strips = n_nodes // zrows
    strip_iters = (n_strips + NUM_SUBCORES - 1) // NUM_SUBCORES

    @pl.loop(0, zrows)
    def _(zr):
        @pl.loop(0, d // LANES)
        def _(c):
            zero_v[zr, pl.ds(c * LANES, LANES)] = jnp.zeros((LANES,), jnp.float32)

    @pl.loop(0, strip_iters)
    def _(t):
        strip = sid + t * NUM_SUBCORES

        @pl.when(strip < n_strips)
        def _():
            pltpu.sync_copy(zero_v, acc.at[pl.ds(strip * zrows, zrows)])

    plsc.subcore_barrier()

    max_iters = (n_chunks_per_core + NUM_SUBCORES - 1) // NUM_SUBCORES

    def edge_base(chunk):
        return (cid * n_chunks_per_core + chunk) * CHUNK

    def issue_idx(chunk, h):
        base = edge_base(chunk)
        pltpu.async_copy(src_hbm.at[pl.ds(base, CHUNK)], src_v.at[h], isem)
        pltpu.async_copy(typ_hbm.at[pl.ds(base, CHUNK)], typ_v.at[h], isem)
        pltpu.async_copy(dst_hbm.at[pl.ds(base, CHUNK)], dst_v.at[h], isem)

    def wait_idx_compute_key_issue_gather(chunk, h):
        base = edge_base(chunk)
        pltpu.make_async_copy(src_hbm.at[pl.ds(base, CHUNK)], src_v.at[h], isem).wait()
        pltpu.make_async_copy(typ_hbm.at[pl.ds(base, CHUNK)], typ_v.at[h], isem).wait()
        pltpu.make_async_copy(dst_hbm.at[pl.ds(base, CHUNK)], dst_v.at[h], isem).wait()

        @pl.loop(0, CHUNK // LANES)
        def _(i):
            sl = pl.ds(i * LANES, LANES)
            key_v[h, sl] = typ_v[h, sl] * n_nodes + src_v[h, sl]

        pltpu.async_copy(p_hbm.at[key_v.at[h]], rows_v.at[h], gsem)

    def stage(c_cur, h_cur, h_nxt):
        c_nxt = c_cur + NUM_SUBCORES

        @pl.when(c_nxt < n_chunks_per_core)
        def _():
            issue_idx(c_nxt, h_nxt)

        @pl.when(c_cur < n_chunks_per_core)
        def _():
            pltpu.make_async_copy(p_hbm.at[key_v.at[h_cur]],
                                  rows_v.at[h_cur], gsem).wait()

        @pl.when(c_nxt < n_chunks_per_core)
        def _():
            wait_idx_compute_key_issue_gather(c_nxt, h_nxt)

        @pl.when(c_cur < n_chunks_per_core)
        def _():
            pltpu.sync_copy(rows_v.at[h_cur], acc.at[dst_v.at[h_cur]], add=True)

    issue_idx(sid, 0)
    wait_idx_compute_key_issue_gather(sid, 0)

    @pl.loop(0, (max_iters + 1) // 2)
    def _(j):
        stage(sid + (2 * j) * NUM_SUBCORES, 0, 1)
        stage(sid + (2 * j + 1) * NUM_SUBCORES, 1, 0)

    plsc.subcore_barrier()

    @pl.loop(0, strip_iters)
    def _(t):
        strip = sid + t * NUM_SUBCORES

        @pl.when(strip < n_strips)
        def _():
            r0 = strip * zrows
            pltpu.sync_copy(acc.at[pl.ds(r0, zrows)],
                            out_hbm.at[cid, pl.ds(r0, zrows)])


def kernel(input, edge_index, edge_type, self_loop_W, self_loop_b,
           linear_W, linear_b):
    n, d = input.shape
    e = edge_index.shape[0]
    r = linear_W.shape[0] // d

    wl_t = linear_W.T
    ws_t = self_loop_W.T
    bl = linear_b.reshape(r, d)
    bs = self_loop_b.reshape(1, d)
    src = edge_index[:, 0].astype(jnp.int32)
    dst = edge_index[:, 1].astype(jnp.int32)
    typ = edge_type.astype(jnp.int32)

    row_blk = 2000
    grid = (n // row_blk,)
    nb = n // row_blk
    p4 = pl.pallas_call(
        _proj_body,
        grid=(nb, r),
        in_specs=[
            pl.BlockSpec((row_blk, d), lambda i, rr: (i, 0)),
            pl.BlockSpec((d, d), lambda i, rr: (0, rr)),
            pl.BlockSpec((r, d), lambda i, rr: (0, 0)),
        ],
        out_specs=pl.BlockSpec((row_blk, d), lambda i, rr: (rr * nb + i, 0)),
        out_shape=jax.ShapeDtypeStruct((n * r, d), jnp.float32),
    )(input, wl_t, bl)

    hidden = pl.pallas_call(
        _hidden_body,
        grid=grid,
        in_specs=[
            pl.BlockSpec((row_blk, d), lambda i: (i, 0)),
            pl.BlockSpec((d, d), lambda i: (0, 0)),
            pl.BlockSpec((1, d), lambda i: (0, 0)),
        ],
        out_specs=pl.BlockSpec((row_blk, d), lambda i: (i, 0)),
        out_shape=jax.ShapeDtypeStruct((n, d), jnp.float32),
    )(input, ws_t, bs)

    n_chunks_per_core = e // (NUM_SC_CORES * CHUNK)
    mesh = plsc.VectorSubcoreMesh(core_axis_name="c", subcore_axis_name="s")
    sc_fn = pl.kernel(
        functools.partial(_sc_body, n, d, r, n_chunks_per_core),
        out_type=jax.ShapeDtypeStruct((NUM_SC_CORES, n, d), jnp.float32),
        mesh=mesh,
        scratch_types=[
            pltpu.VMEM_SHARED((n, d), jnp.float32),
            pltpu.VMEM((2, CHUNK), jnp.int32),
            pltpu.VMEM((2, CHUNK), jnp.int32),
            pltpu.VMEM((2, CHUNK), jnp.int32),
            pltpu.VMEM((2, CHUNK), jnp.int32),
            pltpu.VMEM((2, CHUNK, d), jnp.float32),
            pltpu.VMEM((80, d), jnp.float32),
            pltpu.SemaphoreType.DMA,
            pltpu.SemaphoreType.DMA,
        ],
    )
    acc = sc_fn(p4, src, typ, dst)

    out = pl.pallas_call(
        _combine_body,
        grid=grid,
        in_specs=[
            pl.BlockSpec((row_blk, d), lambda i: (i, 0)),
            pl.BlockSpec((NUM_SC_CORES, row_blk, d), lambda i: (0, i, 0)),
        ],
        out_specs=pl.BlockSpec((row_blk, d), lambda i: (i, 0)),
        out_shape=jax.ShapeDtypeStruct((n, d), jnp.float32),
    )(hidden, acc)
    return out


# speedup vs baseline: 1.7580x; 1.0726x over previous
import functools

import jax
import jax.numpy as jnp
from jax import lax
from jax.experimental import pallas as pl
from jax.experimental.pallas import tpu as pltpu
from jax.experimental.pallas import tpu_sc as plsc

LANES = 16
NUM_SC_CORES = 2
NUM_SUBCORES = 16
CHUNK = 128


def _proj_body(x_ref, wl_ref, bl_ref, p_ref):
    rr = pl.program_id(0)
    n_rel = bl_ref.shape[0]
    bias = bl_ref[0:1, :]
    for k in range(1, n_rel):
        bias = jnp.where(rr == k, bl_ref[k:k + 1, :], bias)
    p_ref[...] = (
        jnp.dot(x_ref[...], wl_ref[...], preferred_element_type=jnp.float32)
        + bias
    )


def _hidden_body(x_ref, ws_ref, bs_ref, h_ref):
    h_ref[...] = (
        jnp.dot(x_ref[...], ws_ref[...], preferred_element_type=jnp.float32)
        + bs_ref[...]
    )


def _combine_body(h_ref, a_ref, o_ref):
    a = a_ref[...]
    o_ref[...] = jnp.maximum(h_ref[...] + a[0] + a[1], 0.0)


def _sc_body(n_nodes, d, n_relations, n_chunks_per_core, p_hbm, src_hbm,
             typ_hbm, dst_hbm, out_hbm, acc, src_v, typ_v, dst_v, key_v,
             rows_v, zero_v, gsem, isem):
    cid = lax.axis_index("c")
    sid = lax.axis_index("s")

    zrows = zero_v.shape[0]
    n_strips = n_nodes // zrows
    strip_iters = (n_strips + NUM_SUBCORES - 1) // NUM_SUBCORES

    @pl.loop(0, zrows)
    def _(zr):
        @pl.loop(0, d // LANES)
        def _(c):
            zero_v[zr, pl.ds(c * LANES, LANES)] = jnp.zeros((LANES,), jnp.float32)

    @pl.loop(0, strip_iters)
    def _(t):
        strip = sid + t * NUM_SUBCORES

        @pl.when(strip < n_strips)
        def _():
            pltpu.sync_copy(zero_v, acc.at[pl.ds(strip * zrows, zrows)])

    plsc.subcore_barrier()

    max_iters = (n_chunks_per_core + NUM_SUBCORES - 1) // NUM_SUBCORES

    def edge_base(chunk):
        return (cid * n_chunks_per_core + chunk) * CHUNK

    def issue_idx(chunk, h):
        base = edge_base(chunk)
        pltpu.async_copy(src_hbm.at[pl.ds(base, CHUNK)], src_v.at[h], isem)
        pltpu.async_copy(typ_hbm.at[pl.ds(base, CHUNK)], typ_v.at[h], isem)
        pltpu.async_copy(dst_hbm.at[pl.ds(base, CHUNK)], dst_v.at[h], isem)

    def wait_idx_compute_key_issue_gather(chunk, h):
        base = edge_base(chunk)
        pltpu.make_async_copy(src_hbm.at[pl.ds(base, CHUNK)], src_v.at[h], isem).wait()
        pltpu.make_async_copy(typ_hbm.at[pl.ds(base, CHUNK)], typ_v.at[h], isem).wait()
        pltpu.make_async_copy(dst_hbm.at[pl.ds(base, CHUNK)], dst_v.at[h], isem).wait()

        @pl.loop(0, CHUNK // LANES)
        def _(i):
            sl = pl.ds(i * LANES, LANES)
            key_v[h, sl] = typ_v[h, sl] * n_nodes + src_v[h, sl]

        pltpu.async_copy(p_hbm.at[key_v.at[h]], rows_v.at[h], gsem)

    def stage(c_cur, h_cur, h_nxt):
        c_nxt = c_cur + NUM_SUBCORES

        @pl.when(c_nxt < n_chunks_per_core)
        def _():
            issue_idx(c_nxt, h_nxt)

        @pl.when(c_cur < n_chunks_per_core)
        def _():
            pltpu.make_async_copy(p_hbm.at[key_v.at[h_cur]],
                                  rows_v.at[h_cur], gsem).wait()

        @pl.when(c_nxt < n_chunks_per_core)
        def _():
            wait_idx_compute_key_issue_gather(c_nxt, h_nxt)

        @pl.when(c_cur < n_chunks_per_core)
        def _():
            pltpu.sync_copy(rows_v.at[h_cur], acc.at[dst_v.at[h_cur]], add=True)

    issue_idx(sid, 0)
    wait_idx_compute_key_issue_gather(sid, 0)

    @pl.loop(0, (max_iters + 1) // 2)
    def _(j):
        stage(sid + (2 * j) * NUM_SUBCORES, 0, 1)
        stage(sid + (2 * j + 1) * NUM_SUBCORES, 1, 0)

    plsc.subcore_barrier()

    @pl.loop(0, strip_iters)
    def _(t):
        strip = sid + t * NUM_SUBCORES

        @pl.when(strip < n_strips)
        def _():
            r0 = strip * zrows
            pltpu.sync_copy(acc.at[pl.ds(r0, zrows)],
                            out_hbm.at[cid, pl.ds(r0, zrows)])


def kernel(input, edge_index, edge_type, self_loop_W, self_loop_b,
           linear_W, linear_b):
    n, d = input.shape
    e = edge_index.shape[0]
    r = linear_W.shape[0] // d

    wl_t = linear_W.T
    ws_t = self_loop_W.T
    bl = linear_b.reshape(r, d)
    bs = self_loop_b.reshape(1, d)
    src = edge_index[:, 0].astype(jnp.int32)
    dst = edge_index[:, 1].astype(jnp.int32)
    typ = edge_type.astype(jnp.int32)

    row_blk = 2000
    grid = (n // row_blk,)
    p4 = pl.pallas_call(
        _proj_body,
        grid=(r,),
        in_specs=[
            pl.BlockSpec((n, d), lambda rr: (0, 0)),
            pl.BlockSpec((d, d), lambda rr: (0, rr)),
            pl.BlockSpec((r, d), lambda rr: (0, 0)),
        ],
        out_specs=pl.BlockSpec((n, d), lambda rr: (rr, 0)),
        out_shape=jax.ShapeDtypeStruct((n * r, d), jnp.float32),
    )(input, wl_t, bl)

    hidden = pl.pallas_call(
        _hidden_body,
        grid=grid,
        in_specs=[
            pl.BlockSpec((row_blk, d), lambda i: (i, 0)),
            pl.BlockSpec((d, d), lambda i: (0, 0)),
            pl.BlockSpec((1, d), lambda i: (0, 0)),
        ],
        out_specs=pl.BlockSpec((row_blk, d), lambda i: (i, 0)),
        out_shape=jax.ShapeDtypeStruct((n, d), jnp.float32),
    )(input, ws_t, bs)

    n_chunks_per_core = e // (NUM_SC_CORES * CHUNK)
    mesh = plsc.VectorSubcoreMesh(core_axis_name="c", subcore_axis_name="s")
    sc_fn = pl.kernel(
        functools.partial(_sc_body, n, d, r, n_chunks_per_core),
        out_type=jax.ShapeDtypeStruct((NUM_SC_CORES, n, d), jnp.float32),
        mesh=mesh,
        scratch_types=[
            pltpu.VMEM_SHARED((n, d), jnp.float32),
            pltpu.VMEM((2, CHUNK), jnp.int32),
            pltpu.VMEM((2, CHUNK), jnp.int32),
            pltpu.VMEM((2, CHUNK), jnp.int32),
            pltpu.VMEM((2, CHUNK), jnp.int32),
            pltpu.VMEM((2, CHUNK, d), jnp.float32),
            pltpu.VMEM((80, d), jnp.float32),
            pltpu.SemaphoreType.DMA,
            pltpu.SemaphoreType.DMA,
        ],
    )
    acc = sc_fn(p4, src, typ, dst)

    out = pl.pallas_call(
        _combine_body,
        grid=grid,
        in_specs=[
            pl.BlockSpec((row_blk, d), lambda i: (i, 0)),
            pl.BlockSpec((NUM_SC_CORES, row_blk, d), lambda i: (0, i, 0)),
        ],
        out_specs=pl.BlockSpec((row_blk, d), lambda i: (i, 0)),
        out_shape=jax.ShapeDtypeStruct((n, d), jnp.float32),
    )(hidden, acc)
    return out


# speedup vs baseline: 1.7600x; 1.0011x over previous
import functools

import jax
import jax.numpy as jnp
from jax import lax
from jax.experimental import pallas as pl
from jax.experimental.pallas import tpu as pltpu
from jax.experimental.pallas import tpu_sc as plsc

LANES = 16
NUM_SC_CORES = 2
NUM_SUBCORES = 16
CHUNK = 128


def _proj_body(x_ref, wl_ref, bl_ref, p_ref):
    rr = pl.program_id(0)
    n_rel = bl_ref.shape[0]
    bias = bl_ref[0:1, :]
    for k in range(1, n_rel):
        bias = jnp.where(rr == k, bl_ref[k:k + 1, :], bias)
    p_ref[...] = (
        jnp.dot(x_ref[...], wl_ref[...], preferred_element_type=jnp.float32)
        + bias
    )


def _hidden_body(x_ref, ws_ref, bs_ref, h_ref):
    h_ref[...] = (
        jnp.dot(x_ref[...], ws_ref[...], preferred_element_type=jnp.float32)
        + bs_ref[...]
    )


def _combine_body(h_ref, a_ref, o_ref):
    a = a_ref[...]
    o_ref[...] = jnp.maximum(h_ref[...] + a[0] + a[1], 0.0)


def _sc_body(n_nodes, d, n_relations, n_chunks_per_core, p_hbm, src_hbm,
             typ_hbm, dst_hbm, out_hbm, acc, src_v, typ_v, dst_v, key_v,
             rows_v, zero_v, gsem, isem, ssem):
    cid = lax.axis_index("c")
    sid = lax.axis_index("s")

    zrows = zero_v.shape[0]
    n_strips = n_nodes // zrows
    strip_iters = (n_strips + NUM_SUBCORES - 1) // NUM_SUBCORES

    @pl.loop(0, zrows)
    def _(zr):
        @pl.loop(0, d // LANES)
        def _(c):
            zero_v[zr, pl.ds(c * LANES, LANES)] = jnp.zeros((LANES,), jnp.float32)

    @pl.loop(0, strip_iters)
    def _(t):
        strip = sid + t * NUM_SUBCORES

        @pl.when(strip < n_strips)
        def _():
            pltpu.sync_copy(zero_v, acc.at[pl.ds(strip * zrows, zrows)])

    plsc.subcore_barrier()

    max_iters = (n_chunks_per_core + NUM_SUBCORES - 1) // NUM_SUBCORES

    def edge_base(chunk):
        return (cid * n_chunks_per_core + chunk) * CHUNK

    def issue_idx(chunk, s):
        base = edge_base(chunk)
        pltpu.async_copy(src_hbm.at[pl.ds(base, CHUNK)], src_v.at[s], isem)
        pltpu.async_copy(typ_hbm.at[pl.ds(base, CHUNK)], typ_v.at[s], isem)
        pltpu.async_copy(dst_hbm.at[pl.ds(base, CHUNK)], dst_v.at[s], isem)

    def wait_idx_compute_key(chunk, s):
        base = edge_base(chunk)
        pltpu.make_async_copy(src_hbm.at[pl.ds(base, CHUNK)], src_v.at[s], isem).wait()
        pltpu.make_async_copy(typ_hbm.at[pl.ds(base, CHUNK)], typ_v.at[s], isem).wait()
        pltpu.make_async_copy(dst_hbm.at[pl.ds(base, CHUNK)], dst_v.at[s], isem).wait()

        @pl.loop(0, CHUNK // LANES)
        def _(i):
            sl = pl.ds(i * LANES, LANES)
            key_v[s, sl] = typ_v[s, sl] * n_nodes + src_v[s, sl]

    def stage(m_t, k):
        hc, hn = k % 2, (k + 1) % 2
        ic, in1, in2 = k % 4, (k + 1) % 4, (k + 2) % 4
        c = sid + m_t * NUM_SUBCORES

        @pl.when(c < n_chunks_per_core)
        def _():
            pltpu.make_async_copy(p_hbm.at[key_v.at[ic]],
                                  rows_v.at[hc], gsem).wait()

        @pl.when((c < n_chunks_per_core) & (m_t >= 1))
        def _():
            pltpu.make_async_copy(rows_v.at[hn],
                                  acc.at[dst_v.at[(k - 1) % 4]], ssem).wait()

        @pl.when(c < n_chunks_per_core)
        def _():
            pltpu.async_copy(rows_v.at[hc], acc.at[dst_v.at[ic]], ssem,
                             add=True)

        @pl.when(c + NUM_SUBCORES < n_chunks_per_core)
        def _():
            wait_idx_compute_key(c + NUM_SUBCORES, in1)
            pltpu.async_copy(p_hbm.at[key_v.at[in1]], rows_v.at[hn], gsem)

        @pl.when(c + 2 * NUM_SUBCORES < n_chunks_per_core)
        def _():
            issue_idx(c + 2 * NUM_SUBCORES, in2)

    issue_idx(sid, 0)
    wait_idx_compute_key(sid, 0)
    pltpu.async_copy(p_hbm.at[key_v.at[0]], rows_v.at[0], gsem)

    @pl.when(sid + NUM_SUBCORES < n_chunks_per_core)
    def _():
        issue_idx(sid + NUM_SUBCORES, 1)

    @pl.loop(0, (max_iters + 3) // 4)
    def _(j):
        stage(4 * j + 0, 0)
        stage(4 * j + 1, 1)
        stage(4 * j + 2, 2)
        stage(4 * j + 3, 3)

    pltpu.make_async_copy(rows_v.at[0], acc.at[dst_v.at[0]], ssem).wait()

    plsc.subcore_barrier()

    @pl.loop(0, strip_iters)
    def _(t):
        strip = sid + t * NUM_SUBCORES

        @pl.when(strip < n_strips)
        def _():
            r0 = strip * zrows
            pltpu.sync_copy(acc.at[pl.ds(r0, zrows)],
                            out_hbm.at[cid, pl.ds(r0, zrows)])


def kernel(input, edge_index, edge_type, self_loop_W, self_loop_b,
           linear_W, linear_b):
    n, d = input.shape
    e = edge_index.shape[0]
    r = linear_W.shape[0] // d

    wl_t = linear_W.T
    ws_t = self_loop_W.T
    bl = linear_b.reshape(r, d)
    bs = self_loop_b.reshape(1, d)
    src = edge_index[:, 0].astype(jnp.int32)
    dst = edge_index[:, 1].astype(jnp.int32)
    typ = edge_type.astype(jnp.int32)

    row_blk = 2000
    grid = (n // row_blk,)
    p4 = pl.pallas_call(
        _proj_body,
        grid=(r,),
        in_specs=[
            pl.BlockSpec((n, d), lambda rr: (0, 0)),
            pl.BlockSpec((d, d), lambda rr: (0, rr)),
            pl.BlockSpec((r, d), lambda rr: (0, 0)),
        ],
        out_specs=pl.BlockSpec((n, d), lambda rr: (rr, 0)),
        out_shape=jax.ShapeDtypeStruct((n * r, d), jnp.float32),
    )(input, wl_t, bl)

    hidden = pl.pallas_call(
        _hidden_body,
        grid=grid,
        in_specs=[
            pl.BlockSpec((row_blk, d), lambda i: (i, 0)),
            pl.BlockSpec((d, d), lambda i: (0, 0)),
            pl.BlockSpec((1, d), lambda i: (0, 0)),
        ],
        out_specs=pl.BlockSpec((row_blk, d), lambda i: (i, 0)),
        out_shape=jax.ShapeDtypeStruct((n, d), jnp.float32),
    )(input, ws_t, bs)

    n_chunks_per_core = e // (NUM_SC_CORES * CHUNK)
    mesh = plsc.VectorSubcoreMesh(core_axis_name="c", subcore_axis_name="s")
    sc_fn = pl.kernel(
        functools.partial(_sc_body, n, d, r, n_chunks_per_core),
        out_type=jax.ShapeDtypeStruct((NUM_SC_CORES, n, d), jnp.float32),
        mesh=mesh,
        scratch_types=[
            pltpu.VMEM_SHARED((n, d), jnp.float32),
            pltpu.VMEM((4, CHUNK), jnp.int32),
            pltpu.VMEM((4, CHUNK), jnp.int32),
            pltpu.VMEM((4, CHUNK), jnp.int32),
            pltpu.VMEM((4, CHUNK), jnp.int32),
            pltpu.VMEM((2, CHUNK, d), jnp.float32),
            pltpu.VMEM((80, d), jnp.float32),
            pltpu.SemaphoreType.DMA,
            pltpu.SemaphoreType.DMA,
            pltpu.SemaphoreType.DMA,
        ],
    )
    acc = sc_fn(p4, src, typ, dst)

    out = pl.pallas_call(
        _combine_body,
        grid=grid,
        in_specs=[
            pl.BlockSpec((row_blk, d), lambda i: (i, 0)),
            pl.BlockSpec((NUM_SC_CORES, row_blk, d), lambda i: (0, i, 0)),
        ],
        out_specs=pl.BlockSpec((row_blk, d), lambda i: (i, 0)),
        out_shape=jax.ShapeDtypeStruct((n, d), jnp.float32),
    )(hidden, acc)
    return out


# speedup vs baseline: 1.7833x; 1.0132x over previous
import functools

import jax
import jax.numpy as jnp
from jax import lax
from jax.experimental import pallas as pl
from jax.experimental.pallas import tpu as pltpu
from jax.experimental.pallas import tpu_sc as plsc

LANES = 16
NUM_SC_CORES = 2
NUM_SUBCORES = 16
CHUNK = 128


def _proj_body(x_ref, wl_ref, bl_ref, p_ref):
    rr = pl.program_id(0)
    n_rel = bl_ref.shape[0]
    bias = bl_ref[0:1, :]
    for k in range(1, n_rel):
        bias = jnp.where(rr == k, bl_ref[k:k + 1, :], bias)
    p_ref[...] = (
        jnp.dot(x_ref[...], wl_ref[...], preferred_element_type=jnp.float32)
        + bias
    )


def _hidden_body(x_ref, ws_ref, bs_ref, h_ref):
    h_ref[...] = (
        jnp.dot(x_ref[...], ws_ref[...], preferred_element_type=jnp.float32)
        + bs_ref[...]
    )


def _combine_body(h_ref, a_ref, o_ref):
    a = a_ref[...]
    o_ref[...] = jnp.maximum(h_ref[...] + a[0] + a[1], 0.0)


def _sc_body(n_nodes, d, n_relations, n_chunks_per_core, p_hbm, src_hbm,
             typ_hbm, dst_hbm, out_hbm, acc, src_v, typ_v, dst_v, key_v,
             rows_v, zero_v, gsem0, gsem1, isem, ssem):
    gsems = (gsem0, gsem1)
    cid = lax.axis_index("c")
    sid = lax.axis_index("s")

    zrows = zero_v.shape[0]
    n_strips = n_nodes // zrows
    strip_iters = (n_strips + NUM_SUBCORES - 1) // NUM_SUBCORES

    @pl.loop(0, zrows)
    def _(zr):
        @pl.loop(0, d // LANES)
        def _(c):
            zero_v[zr, pl.ds(c * LANES, LANES)] = jnp.zeros((LANES,), jnp.float32)

    @pl.loop(0, strip_iters)
    def _(t):
        strip = sid + t * NUM_SUBCORES

        @pl.when(strip < n_strips)
        def _():
            pltpu.sync_copy(zero_v, acc.at[pl.ds(strip * zrows, zrows)])

    plsc.subcore_barrier()

    max_iters = (n_chunks_per_core + NUM_SUBCORES - 1) // NUM_SUBCORES

    def edge_base(chunk):
        return (cid * n_chunks_per_core + chunk) * CHUNK

    def issue_idx(chunk, s):
        base = edge_base(chunk)
        pltpu.async_copy(src_hbm.at[pl.ds(base, CHUNK)], src_v.at[s], isem)
        pltpu.async_copy(typ_hbm.at[pl.ds(base, CHUNK)], typ_v.at[s], isem)
        pltpu.async_copy(dst_hbm.at[pl.ds(base, CHUNK)], dst_v.at[s], isem)

    def wait_idx_compute_key(chunk, s):
        base = edge_base(chunk)
        pltpu.make_async_copy(src_hbm.at[pl.ds(base, CHUNK)], src_v.at[s], isem).wait()
        pltpu.make_async_copy(typ_hbm.at[pl.ds(base, CHUNK)], typ_v.at[s], isem).wait()
        pltpu.make_async_copy(dst_hbm.at[pl.ds(base, CHUNK)], dst_v.at[s], isem).wait()

        @pl.loop(0, CHUNK // LANES)
        def _(i):
            sl = pl.ds(i * LANES, LANES)
            key_v[s, sl] = typ_v[s, sl] * n_nodes + src_v[s, sl]

    half = CHUNK // 2

    def issue_gather(s, h):
        pltpu.async_copy(p_hbm.at[key_v.at[s, pl.ds(0, half)]],
                         rows_v.at[h, pl.ds(0, half)], gsems[0])
        pltpu.async_copy(p_hbm.at[key_v.at[s, pl.ds(half, half)]],
                         rows_v.at[h, pl.ds(half, half)], gsems[1])

    def wait_gather(s, h):
        pltpu.make_async_copy(p_hbm.at[key_v.at[s, pl.ds(0, half)]],
                              rows_v.at[h, pl.ds(0, half)], gsems[0]).wait()
        pltpu.make_async_copy(p_hbm.at[key_v.at[s, pl.ds(half, half)]],
                              rows_v.at[h, pl.ds(half, half)], gsems[1]).wait()

    def stage(m_t, k):
        hc, hn = k % 2, (k + 1) % 2
        ic, in1, in2 = k % 4, (k + 1) % 4, (k + 2) % 4
        c = sid + m_t * NUM_SUBCORES

        @pl.when(c < n_chunks_per_core)
        def _():
            wait_gather(ic, hc)

        @pl.when((c < n_chunks_per_core) & (m_t >= 1))
        def _():
            pltpu.make_async_copy(rows_v.at[hn],
                                  acc.at[dst_v.at[(k - 1) % 4]], ssem).wait()

        @pl.when(c < n_chunks_per_core)
        def _():
            pltpu.async_copy(rows_v.at[hc], acc.at[dst_v.at[ic]], ssem,
                             add=True)

        @pl.when(c + NUM_SUBCORES < n_chunks_per_core)
        def _():
            wait_idx_compute_key(c + NUM_SUBCORES, in1)
            issue_gather(in1, hn)

        @pl.when(c + 2 * NUM_SUBCORES < n_chunks_per_core)
        def _():
            issue_idx(c + 2 * NUM_SUBCORES, in2)

    issue_idx(sid, 0)
    wait_idx_compute_key(sid, 0)
    issue_gather(0, 0)

    @pl.when(sid + NUM_SUBCORES < n_chunks_per_core)
    def _():
        issue_idx(sid + NUM_SUBCORES, 1)

    @pl.loop(0, (max_iters + 3) // 4)
    def _(j):
        for k in range(4):
            stage(4 * j + k, k)

    pltpu.make_async_copy(rows_v.at[0], acc.at[dst_v.at[0]], ssem).wait()

    plsc.subcore_barrier()

    @pl.loop(0, strip_iters)
    def _(t):
        strip = sid + t * NUM_SUBCORES

        @pl.when(strip < n_strips)
        def _():
            r0 = strip * zrows
            pltpu.sync_copy(acc.at[pl.ds(r0, zrows)],
                            out_hbm.at[cid, pl.ds(r0, zrows)])


def kernel(input, edge_index, edge_type, self_loop_W, self_loop_b,
           linear_W, linear_b):
    n, d = input.shape
    e = edge_index.shape[0]
    r = linear_W.shape[0] // d

    wl_t = linear_W.T
    ws_t = self_loop_W.T
    bl = linear_b.reshape(r, d)
    bs = self_loop_b.reshape(1, d)
    src = edge_index[:, 0].astype(jnp.int32)
    dst = edge_index[:, 1].astype(jnp.int32)
    typ = edge_type.astype(jnp.int32)

    row_blk = 2000
    grid = (n // row_blk,)
    p4 = pl.pallas_call(
        _proj_body,
        grid=(r,),
        in_specs=[
            pl.BlockSpec((n, d), lambda rr: (0, 0)),
            pl.BlockSpec((d, d), lambda rr: (0, rr)),
            pl.BlockSpec((r, d), lambda rr: (0, 0)),
        ],
        out_specs=pl.BlockSpec((n, d), lambda rr: (rr, 0)),
        out_shape=jax.ShapeDtypeStruct((n * r, d), jnp.float32),
    )(input, wl_t, bl)

    hidden = pl.pallas_call(
        _hidden_body,
        grid=grid,
        in_specs=[
            pl.BlockSpec((row_blk, d), lambda i: (i, 0)),
            pl.BlockSpec((d, d), lambda i: (0, 0)),
            pl.BlockSpec((1, d), lambda i: (0, 0)),
        ],
        out_specs=pl.BlockSpec((row_blk, d), lambda i: (i, 0)),
        out_shape=jax.ShapeDtypeStruct((n, d), jnp.float32),
    )(input, ws_t, bs)

    n_chunks_per_core = e // (NUM_SC_CORES * CHUNK)
    mesh = plsc.VectorSubcoreMesh(core_axis_name="c", subcore_axis_name="s")
    sc_fn = pl.kernel(
        functools.partial(_sc_body, n, d, r, n_chunks_per_core),
        out_type=jax.ShapeDtypeStruct((NUM_SC_CORES, n, d), jnp.float32),
        mesh=mesh,
        scratch_types=[
            pltpu.VMEM_SHARED((n, d), jnp.float32),
            pltpu.VMEM((4, CHUNK), jnp.int32),
            pltpu.VMEM((4, CHUNK), jnp.int32),
            pltpu.VMEM((4, CHUNK), jnp.int32),
            pltpu.VMEM((4, CHUNK), jnp.int32),
            pltpu.VMEM((2, CHUNK, d), jnp.float32),
            pltpu.VMEM((80, d), jnp.float32),
            pltpu.SemaphoreType.DMA,
            pltpu.SemaphoreType.DMA,
            pltpu.SemaphoreType.DMA,
            pltpu.SemaphoreType.DMA,
        ],
    )
    acc = sc_fn(p4, src, typ, dst)

    out = pl.pallas_call(
        _combine_body,
        grid=grid,
        in_specs=[
            pl.BlockSpec((row_blk, d), lambda i: (i, 0)),
            pl.BlockSpec((NUM_SC_CORES, row_blk, d), lambda i: (0, i, 0)),
        ],
        out_specs=pl.BlockSpec((row_blk, d), lambda i: (i, 0)),
        out_shape=jax.ShapeDtypeStruct((n, d), jnp.float32),
    )(hidden, acc)
    return out


# speedup vs baseline: 1.7989x; 1.0088x over previous
import functools

import jax
import jax.numpy as jnp
from jax import lax
from jax.experimental import pallas as pl
from jax.experimental.pallas import tpu as pltpu
from jax.experimental.pallas import tpu_sc as plsc

LANES = 16
NUM_SC_CORES = 2
NUM_SUBCORES = 16
CHUNK = 128


def _proj_body(x_ref, wl_ref, bl_ref, p_ref):
    rr = pl.program_id(0)
    n_rel = bl_ref.shape[0]
    bias = bl_ref[0:1, :]
    for k in range(1, n_rel):
        bias = jnp.where(rr == k, bl_ref[k:k + 1, :], bias)
    p_ref[...] = (
        jnp.dot(x_ref[...], wl_ref[...], preferred_element_type=jnp.float32)
        + bias
    )


def _hidden_body(x_ref, ws_ref, bs_ref, h_ref):
    h_ref[...] = (
        jnp.dot(x_ref[...], ws_ref[...], preferred_element_type=jnp.float32)
        + bs_ref[...]
    )


def _combine_body(h_ref, a_ref, o_ref):
    a = a_ref[...]
    o_ref[...] = jnp.maximum(h_ref[...] + a[0] + a[1], 0.0)


def _sc_body(n_nodes, d, n_relations, n_chunks_per_core, p_hbm, src_hbm,
             typ_hbm, dst_hbm, out_hbm, acc, src_v, typ_v, dst_v, key_v,
             rows_v, zero_v, gsem0, gsem1, isem, ssem):
    gsems = (gsem0, gsem1)
    cid = lax.axis_index("c")
    sid = lax.axis_index("s")

    zrows = zero_v.shape[0]
    n_strips = n_nodes // zrows
    strip_iters = (n_strips + NUM_SUBCORES - 1) // NUM_SUBCORES

    max_iters = (n_chunks_per_core + NUM_SUBCORES - 1) // NUM_SUBCORES

    def edge_base(chunk):
        return (cid * n_chunks_per_core + chunk) * CHUNK

    def issue_idx(chunk, s):
        base = edge_base(chunk)
        pltpu.async_copy(src_hbm.at[pl.ds(base, CHUNK)], src_v.at[s], isem)
        pltpu.async_copy(typ_hbm.at[pl.ds(base, CHUNK)], typ_v.at[s], isem)
        pltpu.async_copy(dst_hbm.at[pl.ds(base, CHUNK)], dst_v.at[s], isem)

    def wait_idx_compute_key(chunk, s):
        base = edge_base(chunk)
        pltpu.make_async_copy(src_hbm.at[pl.ds(base, CHUNK)], src_v.at[s], isem).wait()
        pltpu.make_async_copy(typ_hbm.at[pl.ds(base, CHUNK)], typ_v.at[s], isem).wait()
        pltpu.make_async_copy(dst_hbm.at[pl.ds(base, CHUNK)], dst_v.at[s], isem).wait()

        @pl.loop(0, CHUNK // LANES)
        def _(i):
            sl = pl.ds(i * LANES, LANES)
            key_v[s, sl] = typ_v[s, sl] * n_nodes + src_v[s, sl]

    half = CHUNK // 2

    def issue_gather(s, h):
        pltpu.async_copy(p_hbm.at[key_v.at[s, pl.ds(0, half)]],
                         rows_v.at[h, pl.ds(0, half)], gsems[0])
        pltpu.async_copy(p_hbm.at[key_v.at[s, pl.ds(half, half)]],
                         rows_v.at[h, pl.ds(half, half)], gsems[1])

    def wait_gather(s, h):
        pltpu.make_async_copy(p_hbm.at[key_v.at[s, pl.ds(0, half)]],
                              rows_v.at[h, pl.ds(0, half)], gsems[0]).wait()
        pltpu.make_async_copy(p_hbm.at[key_v.at[s, pl.ds(half, half)]],
                              rows_v.at[h, pl.ds(half, half)], gsems[1]).wait()

    def stage(m_t, k):
        hc, hn = k % 2, (k + 1) % 2
        ic, in1, in2 = k % 4, (k + 1) % 4, (k + 2) % 4
        c = sid + m_t * NUM_SUBCORES

        @pl.when(c < n_chunks_per_core)
        def _():
            wait_gather(ic, hc)

        @pl.when((c < n_chunks_per_core) & (m_t >= 1))
        def _():
            pltpu.make_async_copy(rows_v.at[hn],
                                  acc.at[dst_v.at[(k - 1) % 4]], ssem).wait()

        @pl.when(c < n_chunks_per_core)
        def _():
            pltpu.async_copy(rows_v.at[hc], acc.at[dst_v.at[ic]], ssem,
                             add=True)

        @pl.when(c + NUM_SUBCORES < n_chunks_per_core)
        def _():
            wait_idx_compute_key(c + NUM_SUBCORES, in1)
            issue_gather(in1, hn)

        @pl.when(c + 2 * NUM_SUBCORES < n_chunks_per_core)
        def _():
            issue_idx(c + 2 * NUM_SUBCORES, in2)

    issue_idx(sid, 0)
    wait_idx_compute_key(sid, 0)
    issue_gather(0, 0)

    @pl.when(sid + NUM_SUBCORES < n_chunks_per_core)
    def _():
        issue_idx(sid + NUM_SUBCORES, 1)

    @pl.loop(0, zrows)
    def _(zr):
        @pl.loop(0, d // LANES)
        def _(c):
            zero_v[zr, pl.ds(c * LANES, LANES)] = jnp.zeros((LANES,), jnp.float32)

    @pl.loop(0, strip_iters)
    def _(t):
        strip = sid + t * NUM_SUBCORES

        @pl.when(strip < n_strips)
        def _():
            pltpu.sync_copy(zero_v, acc.at[pl.ds(strip * zrows, zrows)])

    plsc.subcore_barrier()

    @pl.loop(0, (max_iters + 3) // 4)
    def _(j):
        for k in range(4):
            stage(4 * j + k, k)

    pltpu.make_async_copy(rows_v.at[0], acc.at[dst_v.at[0]], ssem).wait()

    plsc.subcore_barrier()

    @pl.loop(0, strip_iters)
    def _(t):
        strip = sid + t * NUM_SUBCORES

        @pl.when(strip < n_strips)
        def _():
            r0 = strip * zrows
            pltpu.sync_copy(acc.at[pl.ds(r0, zrows)],
                            out_hbm.at[cid, pl.ds(r0, zrows)])


def kernel(input, edge_index, edge_type, self_loop_W, self_loop_b,
           linear_W, linear_b):
    n, d = input.shape
    e = edge_index.shape[0]
    r = linear_W.shape[0] // d

    wl_t = linear_W.T
    ws_t = self_loop_W.T
    bl = linear_b.reshape(r, d)
    bs = self_loop_b.reshape(1, d)
    src = edge_index[:, 0].astype(jnp.int32)
    dst = edge_index[:, 1].astype(jnp.int32)
    typ = edge_type.astype(jnp.int32)

    row_blk = 2000
    grid = (n // row_blk,)
    p4 = pl.pallas_call(
        _proj_body,
        grid=(r,),
        in_specs=[
            pl.BlockSpec((n, d), lambda rr: (0, 0)),
            pl.BlockSpec((d, d), lambda rr: (0, rr)),
            pl.BlockSpec((r, d), lambda rr: (0, 0)),
        ],
        out_specs=pl.BlockSpec((n, d), lambda rr: (rr, 0)),
        out_shape=jax.ShapeDtypeStruct((n * r, d), jnp.float32),
    )(input, wl_t, bl)

    hidden = pl.pallas_call(
        _hidden_body,
        grid=grid,
        in_specs=[
            pl.BlockSpec((row_blk, d), lambda i: (i, 0)),
            pl.BlockSpec((d, d), lambda i: (0, 0)),
            pl.BlockSpec((1, d), lambda i: (0, 0)),
        ],
        out_specs=pl.BlockSpec((row_blk, d), lambda i: (i, 0)),
        out_shape=jax.ShapeDtypeStruct((n, d), jnp.float32),
    )(input, ws_t, bs)

    n_chunks_per_core = e // (NUM_SC_CORES * CHUNK)
    mesh = plsc.VectorSubcoreMesh(core_axis_name="c", subcore_axis_name="s")
    sc_fn = pl.kernel(
        functools.partial(_sc_body, n, d, r, n_chunks_per_core),
        out_type=jax.ShapeDtypeStruct((NUM_SC_CORES, n, d), jnp.float32),
        mesh=mesh,
        scratch_types=[
            pltpu.VMEM_SHARED((n, d), jnp.float32),
            pltpu.VMEM((4, CHUNK), jnp.int32),
            pltpu.VMEM((4, CHUNK), jnp.int32),
            pltpu.VMEM((4, CHUNK), jnp.int32),
            pltpu.VMEM((4, CHUNK), jnp.int32),
            pltpu.VMEM((2, CHUNK, d), jnp.float32),
            pltpu.VMEM((80, d), jnp.float32),
            pltpu.SemaphoreType.DMA,
            pltpu.SemaphoreType.DMA,
            pltpu.SemaphoreType.DMA,
            pltpu.SemaphoreType.DMA,
        ],
    )
    acc = sc_fn(p4, src, typ, dst)

    out = pl.pallas_call(
        _combine_body,
        grid=grid,
        in_specs=[
            pl.BlockSpec((row_blk, d), lambda i: (i, 0)),
            pl.BlockSpec((NUM_SC_CORES, row_blk, d), lambda i: (0, i, 0)),
        ],
        out_specs=pl.BlockSpec((row_blk, d), lambda i: (i, 0)),
        out_shape=jax.ShapeDtypeStruct((n, d), jnp.float32),
    )(hidden, acc)
    return out
